# baseline (speedup 1.0000x reference)
import functools

import jax
import jax.numpy as jnp
from jax import lax
from jax.experimental import pallas as pl
from jax.experimental.pallas import tpu as pltpu
from jax.experimental.pallas import tpu_sc as plsc

NC = 2
NS = 16
L = 16
NW = NC * NS
K = 32
CH = 128
TBLK = 2048


def _transpose_body(t_ref, o_ref):
    eye = jnp.eye(K, dtype=jnp.float32)
    o_ref[...] = lax.dot_general(
        t_ref[...], eye, (((0,), (0,)), ((), ())),
        preferred_element_type=jnp.float32)


def _to_row_major(table_t):
    n = table_t.shape[1]
    return pl.pallas_call(
        _transpose_body,
        grid=(pl.cdiv(n, TBLK),),
        in_specs=[pl.BlockSpec((K, TBLK), lambda j: (0, j))],
        out_specs=pl.BlockSpec((TBLK, K), lambda j: (j, 0)),
        out_shape=jax.ShapeDtypeStruct((n, K), jnp.float32),
    )(table_t)


def kernel(data_u, data_i, u_table, i_table):
    B = data_u.shape[0]
    bw = B // NW
    mesh = plsc.VectorSubcoreMesh(core_axis_name="c", subcore_axis_name="s")

    @pl.kernel(
        mesh=mesh,
        out_type=jax.ShapeDtypeStruct((B,), jnp.float32),
        scratch_types=[
            pltpu.VMEM((bw,), jnp.int32),
            pltpu.VMEM((bw,), jnp.int32),
            pltpu.VMEM((bw, K), jnp.float32),
            pltpu.VMEM((bw, K), jnp.float32),
            pltpu.VMEM((bw + L,), jnp.float32),
            pltpu.SemaphoreType.DMA,
            pltpu.SemaphoreType.DMA,
        ],
        compiler_params=pltpu.CompilerParams(
            needs_layout_passes=False, use_tc_tiling_on_sc=False),
    )
    def mf(du, di, ut, it, out, idx_u, idx_i, u_rows, i_rows, out_v,
           sem_u, sem_i):
        wid = lax.axis_index("s") * NC + lax.axis_index("c")
        base = wid * bw

        pltpu.sync_copy(du.at[pl.ds(base, bw)], idx_u)
        pltpu.sync_copy(di.at[pl.ds(base, bw)], idx_i)

        copies = []
        for c in range(bw // CH):
            copies.append(pltpu.async_copy(
                ut.at[idx_u.at[pl.ds(c * CH, CH)]],
                u_rows.at[pl.ds(c * CH, CH)], sem_u))
            copies.append(pltpu.async_copy(
                it.at[idx_i.at[pl.ds(c * CH, CH)]],
                i_rows.at[pl.ds(c * CH, CH)], sem_i))
        for cp in copies:
            cp.wait()

        last_lane = lax.iota(jnp.int32, L) == (L - 1)

        @plsc.parallel_loop(0, bw, 1, unroll=8)
        def _(r):
            p = (u_rows[r, pl.ds(0, L)] * i_rows[r, pl.ds(0, L)] +
                 u_rows[r, pl.ds(L, L)] * i_rows[r, pl.ds(L, L)])
            s = plsc.cumsum(p)
            plsc.store_compressed(out_v.at[pl.ds(r, L)], s, mask=last_lane)

        pltpu.sync_copy(out_v.at[pl.ds(0, bw)], out.at[pl.ds(base, bw)])

    u_rm = _to_row_major(u_table.T)
    i_rm = _to_row_major(i_table.T)
    return mf(data_u.astype(jnp.int32), data_i.astype(jnp.int32), u_rm, i_rm)


# speedup vs baseline: 6.8854x; 6.8854x over previous
import jax
import jax.numpy as jnp
from jax import lax
from jax.experimental import pallas as pl
from jax.experimental.pallas import tpu as pltpu
from jax.experimental.pallas import tpu_sc as plsc

NC = 2
NS = 16
L = 16
NW = NC * NS
K = 32
TW = 128
NBUF = 4


def kernel(data_u, data_i, u_table, i_table):
    B = data_u.shape[0]
    bw = B // NW
    mesh = plsc.VectorSubcoreMesh(core_axis_name="c", subcore_axis_name="s")

    @pl.kernel(
        mesh=mesh,
        out_type=jax.ShapeDtypeStruct((B,), jnp.float32),
        scratch_types=[
            pltpu.VMEM((bw + L,), jnp.int32),
            pltpu.VMEM((bw + L,), jnp.int32),
            pltpu.VMEM((NBUF, K, TW), jnp.float32),
            pltpu.VMEM((NBUF, K, TW), jnp.float32),
            pltpu.VMEM((bw + L,), jnp.float32),
            [pltpu.SemaphoreType.DMA] * NBUF,
            [pltpu.SemaphoreType.DMA] * NBUF,
        ],
        compiler_params=pltpu.CompilerParams(needs_layout_passes=False),
    )
    def mf(du, di, ut, it, out, idx_u, idx_i, w_u, w_i, out_v,
           sems_u, sems_i):
        wid = lax.axis_index("s") * NC + lax.axis_index("c")
        base = wid * bw

        pltpu.sync_copy(du.at[pl.ds(base, bw)], idx_u.at[pl.ds(0, bw)])
        pltpu.sync_copy(di.at[pl.ds(base, bw)], idx_i.at[pl.ds(0, bw)])

        def issue(e, s):
            ru = idx_u[pl.ds(e, L)][0]
            ri = idx_i[pl.ds(e, L)][0]
            bu = pl.multiple_of((ru >> 7) * TW, TW)
            bi = pl.multiple_of((ri >> 7) * TW, TW)
            pltpu.async_copy(ut.at[:, pl.ds(bu, TW)], w_u.at[s], sems_u[s])
            pltpu.async_copy(it.at[:, pl.ds(bi, TW)], w_i.at[s], sems_i[s])

        klo = lax.iota(jnp.int32, L)
        khi = klo + L
        last_lane = klo == (L - 1)

        def extract(e, s):
            pltpu.make_async_copy(ut.at[:, pl.ds(0, TW)], w_u.at[s],
                                  sems_u[s]).wait()
            pltpu.make_async_copy(it.at[:, pl.ds(0, TW)], w_i.at[s],
                                  sems_i[s]).wait()
            pu = jnp.broadcast_to(idx_u[pl.ds(e, L)][0] & (TW - 1), (L,))
            pi = jnp.broadcast_to(idx_i[pl.ds(e, L)][0] & (TW - 1), (L,))
            p = (plsc.load_gather(w_u.at[s], [klo, pu]) *
                 plsc.load_gather(w_i.at[s], [klo, pi]) +
                 plsc.load_gather(w_u.at[s], [khi, pu]) *
                 plsc.load_gather(w_i.at[s], [khi, pi]))
            acc = plsc.cumsum(p)
            plsc.store_compressed(out_v.at[pl.ds(e, L)], acc, mask=last_lane)

        for s in range(NBUF):
            issue(s, s)

        def body(g, _):
            e0 = g * NBUF
            for s in range(NBUF):
                extract(e0 + s, s)

                @pl.when(e0 + s + NBUF < bw)
                def _():
                    issue(e0 + s + NBUF, s)
            return 0

        lax.fori_loop(0, bw // NBUF, body, 0)

        pltpu.sync_copy(out_v.at[pl.ds(0, bw)], out.at[pl.ds(base, bw)])

    return mf(data_u.astype(jnp.int32), data_i.astype(jnp.int32),
              u_table.T, i_table.T)


# speedup vs baseline: 7.0742x; 1.0274x over previous
import jax
import jax.numpy as jnp
from jax import lax
from jax.experimental import pallas as pl
from jax.experimental.pallas import tpu as pltpu
from jax.experimental.pallas import tpu_sc as plsc

NC = 2
NS = 16
L = 16
NW = NC * NS
K = 32
TW = 128
NBUF = 8


def kernel(data_u, data_i, u_table, i_table):
    B = data_u.shape[0]
    bw = B // NW
    mesh = plsc.VectorSubcoreMesh(core_axis_name="c", subcore_axis_name="s")

    @pl.kernel(
        mesh=mesh,
        out_type=jax.ShapeDtypeStruct((B,), jnp.float32),
        scratch_types=[
            pltpu.VMEM((bw + L,), jnp.int32),
            pltpu.VMEM((bw + L,), jnp.int32),
            pltpu.VMEM((NBUF, K, TW), jnp.float32),
            pltpu.VMEM((NBUF, K, TW), jnp.float32),
            pltpu.VMEM((bw + L,), jnp.float32),
            [pltpu.SemaphoreType.DMA] * NBUF,
            [pltpu.SemaphoreType.DMA] * NBUF,
        ],
        compiler_params=pltpu.CompilerParams(needs_layout_passes=False),
    )
    def mf(du, di, ut, it, out, idx_u, idx_i, w_u, w_i, out_v,
           sems_u, sems_i):
        wid = lax.axis_index("s") * NC + lax.axis_index("c")
        base = wid * bw

        pltpu.sync_copy(du.at[pl.ds(base, bw)], idx_u.at[pl.ds(0, bw)])
        pltpu.sync_copy(di.at[pl.ds(base, bw)], idx_i.at[pl.ds(0, bw)])

        def issue(e, s):
            ru = idx_u[pl.ds(e, L)][0]
            ri = idx_i[pl.ds(e, L)][0]
            bu = pl.multiple_of((ru >> 7) * TW, TW)
            bi = pl.multiple_of((ri >> 7) * TW, TW)
            pltpu.async_copy(ut.at[:, pl.ds(bu, TW)], w_u.at[s], sems_u[s])
            pltpu.async_copy(it.at[:, pl.ds(bi, TW)], w_i.at[s], sems_i[s])

        klo = lax.iota(jnp.int32, L)
        khi = klo + L
        last_lane = klo == (L - 1)

        def extract(e, s):
            pltpu.make_async_copy(ut.at[:, pl.ds(0, TW)], w_u.at[s],
                                  sems_u[s]).wait()
            pltpu.make_async_copy(it.at[:, pl.ds(0, TW)], w_i.at[s],
                                  sems_i[s]).wait()
            pu = jnp.broadcast_to(idx_u[pl.ds(e, L)][0] & (TW - 1), (L,))
            pi = jnp.broadcast_to(idx_i[pl.ds(e, L)][0] & (TW - 1), (L,))
            p = (plsc.load_gather(w_u.at[s], [klo, pu]) *
                 plsc.load_gather(w_i.at[s], [klo, pi]) +
                 plsc.load_gather(w_u.at[s], [khi, pu]) *
                 plsc.load_gather(w_i.at[s], [khi, pi]))
            acc = plsc.cumsum(p)
            plsc.store_compressed(out_v.at[pl.ds(e, L)], acc, mask=last_lane)

        for s in range(NBUF):
            issue(s, s)

        def body(g, _):
            e0 = g * NBUF
            for s in range(NBUF):
                extract(e0 + s, s)

                @pl.when(e0 + s + NBUF < bw)
                def _():
                    issue(e0 + s + NBUF, s)
            return 0

        lax.fori_loop(0, bw // NBUF, body, 0)

        pltpu.sync_copy(out_v.at[pl.ds(0, bw)], out.at[pl.ds(base, bw)])

    return mf(data_u.astype(jnp.int32), data_i.astype(jnp.int32),
              u_table.T, i_table.T)
